# baseline (speedup 1.0000x reference)
import functools

import jax
import jax.numpy as jnp
from jax import lax
from jax.experimental import pallas as pl
from jax.experimental.pallas import tpu as pltpu
from jax.experimental.pallas import tpu_sc as plsc

N = 16384
C = 1000
RATE = 0.8
K = min(N, int(N * RATE))

NT = 0
NS = N - NT
BR = 1024
NBT = NT // BR
NBT2 = NBT // 2

SC_NC = 2
SC_NSUB = 16
SC_NW = SC_NC * SC_NSUB
ROWS_W = NS // SC_NW
GROUPS = ROWS_W // SC_NSUB


def _sc_body(x_hbm, tgt_hbm, s_out, tv_out, stage0, stage1, tbuf,
             s_buf, tv_buf, sem):
    c = lax.axis_index("c")
    sub = lax.axis_index("s")
    wid = sub * SC_NC + c
    base = wid * ROWS_W
    lanes = lax.iota(jnp.int32, 16)
    row_off = lanes * C
    stages = (stage0, stage1)

    pltpu.sync_copy(tgt_hbm.at[pl.ds(base, ROWS_W)], tbuf)

    def fire(g):
        rbase = base + g * SC_NSUB
        stage = stages[g % 2]
        return pltpu.async_copy(
            x_hbm.at[pl.ds(rbase, SC_NSUB)], stage, sem)

    handle = fire(0)
    for g in range(GROUPS):
        nxt = fire(g + 1) if g + 1 < GROUPS else None
        handle.wait()
        handle = nxt
        stage = stages[g % 2]

        LANES_PER_IT = 8

        def col_body(j, accs):
            new = []
            for u in range(LANES_PER_IT):
                cols = jnp.full((16,), j * LANES_PER_IT + u, jnp.int32)
                vals = plsc.load_gather(stage, [lanes, cols])
                new.append(accs[u] + jnp.exp(vals))
            return tuple(new)

        accs = lax.fori_loop(
            0, C // LANES_PER_IT, col_body,
            tuple(jnp.zeros((16,), jnp.float32) for _ in range(LANES_PER_IT)))
        acc = accs[0]
        for u in range(1, LANES_PER_IT):
            acc = acc + accs[u]
        tvec = tbuf[pl.ds(g * SC_NSUB, SC_NSUB)]
        tcol = jnp.where(tvec == -1, 0, tvec)
        tv = plsc.load_gather(stage, [lanes, tcol])
        s_buf[pl.ds(g * SC_NSUB, SC_NSUB)] = acc
        tv_buf[pl.ds(g * SC_NSUB, SC_NSUB)] = tv

    pltpu.sync_copy(s_buf, s_out.at[pl.ds(wid * ROWS_W, ROWS_W)])
    pltpu.sync_copy(tv_buf, tv_out.at[pl.ds(wid * ROWS_W, ROWS_W)])


@functools.partial(
    pl.kernel,
    out_type=(
        jax.ShapeDtypeStruct((NS,), jnp.float32),
        jax.ShapeDtypeStruct((NS,), jnp.float32),
    ),
    mesh=plsc.VectorSubcoreMesh(core_axis_name="c", subcore_axis_name="s"),
    compiler_params=pltpu.CompilerParams(
        needs_layout_passes=False, use_tc_tiling_on_sc=False),
    scratch_types=[
        pltpu.VMEM((SC_NSUB, C), jnp.float32),
        pltpu.VMEM((SC_NSUB, C), jnp.float32),
        pltpu.VMEM((ROWS_W,), jnp.int32),
        pltpu.VMEM((ROWS_W,), jnp.float32),
        pltpu.VMEM((ROWS_W,), jnp.float32),
        pltpu.SemaphoreType.DMA,
    ],
)
def _sc_ce(x_hbm, tgt_hbm, s_out, tv_out, stage0, stage1, tbuf,
           s_buf, tv_buf, sem):
    _sc_body(x_hbm, tgt_hbm, s_out, tv_out, stage0, stage1, tbuf,
             s_buf, tv_buf, sem)


def _sel_body(s_ref, tv_ref, tg_ref, out_ref):
    ls = jnp.maximum(jnp.log(s_ref[...]) - tv_ref[...], 0.0)
    ls = jnp.where(tg_ref[...] == -1, 0.0, ls)
    u = lax.bitcast_convert_type(ls, jnp.int32)

    def body(_, lo_hi):
        lo, hi = lo_hi
        mid = lo + ((hi - lo + 1) >> 1)
        cnt = jnp.sum((u >= mid).astype(jnp.int32))
        ge = cnt >= K
        return jnp.where(ge, mid, lo), jnp.where(ge, hi, mid - 1)

    lo, _ = lax.fori_loop(0, 31, body, (jnp.int32(0), jnp.int32(0x7F7FFFFF)))
    t_kth = lax.bitcast_convert_type(lo, jnp.float32)
    c_gt = jnp.sum((u > lo).astype(jnp.int32))
    s_gt = jnp.sum(jnp.where(u > lo, ls, 0.0))
    out_ref[0, 0] = (s_gt + (K - c_gt).astype(jnp.float32) * t_kth) / K


@jax.jit
def _select(s_sc, tv_sc, tgt_sc):
    out = pl.pallas_call(
        _sel_body,
        out_specs=pl.BlockSpec(memory_space=pltpu.SMEM),
        out_shape=jax.ShapeDtypeStruct((1, 1), jnp.float32),
    )(s_sc, tv_sc, tgt_sc)
    return out[0, 0]


def kernel(cls_pred, cls_target):
    tgt = cls_target.astype(jnp.int32)
    s_sc, tv_sc = _sc_ce(cls_pred, tgt)
    return _select(s_sc.reshape(NS // BR, BR),
                   tv_sc.reshape(NS // BR, BR),
                   tgt.reshape(NS // BR, BR))


# speedup vs baseline: 2.1929x; 2.1929x over previous
import jax
import jax.numpy as jnp
from jax.experimental import pallas as pl
from jax.experimental.pallas import tpu as pltpu

N = 16384
C = 1000
RATE = 0.8
K = min(N, int(N * RATE))
BR = 1024
NB = N // BR
NB2 = NB // 2


def _ce_rows(x, t):
    col = jax.lax.broadcasted_iota(jnp.int32, (BR, C), 1)
    onehot = col == t[:, None]
    s = jnp.sum(jnp.exp(x), axis=1)
    tval = jnp.sum(jnp.where(onehot, x, 0.0), axis=1)
    loss = jnp.maximum(jnp.log(s) - tval, 0.0)
    return jnp.where(t == -1, 0.0, loss)


def _ohem_body(x0_ref, x1_ref, t0_ref, t1_ref, out_ref, loss_scr):
    i = pl.program_id(0)
    loss_scr[i, :] = _ce_rows(x0_ref[...], t0_ref[0, 0, :])
    loss_scr[i + NB2, :] = _ce_rows(x1_ref[...], t1_ref[0, 0, :])

    @pl.when(i == NB2 - 1)
    def _select():
        v = loss_scr[...]
        u = jax.lax.bitcast_convert_type(v, jnp.int32)

        def body(_, lo_hi):
            lo, hi = lo_hi
            mid = lo + ((hi - lo + 1) >> 1)
            cnt = jnp.sum((u >= mid).astype(jnp.int32))
            ge = cnt >= K
            return jnp.where(ge, mid, lo), jnp.where(ge, hi, mid - 1)

        lo, _ = jax.lax.fori_loop(
            0, 31, body, (jnp.int32(0), jnp.int32(0x7F7FFFFF)))
        t_kth = jax.lax.bitcast_convert_type(lo, jnp.float32)
        gt = u > lo
        c_gt = jnp.sum(gt.astype(jnp.int32))
        s_gt = jnp.sum(jnp.where(gt, v, 0.0))
        out_ref[0, 0] = (s_gt + (K - c_gt).astype(jnp.float32) * t_kth) / K


@jax.jit
def _ohem(cls_pred, tgt3):
    out = pl.pallas_call(
        _ohem_body,
        grid=(NB2,),
        in_specs=[
            pl.BlockSpec((BR, C), lambda i: (i, 0)),
            pl.BlockSpec((BR, C), lambda i: (i + NB2, 0)),
            pl.BlockSpec((1, 1, BR), lambda i: (i, 0, 0)),
            pl.BlockSpec((1, 1, BR), lambda i: (i + NB2, 0, 0)),
        ],
        out_specs=pl.BlockSpec(
            (1, 1), lambda i: (0, 0), memory_space=pltpu.SMEM),
        out_shape=jax.ShapeDtypeStruct((1, 1), jnp.float32),
        scratch_shapes=[pltpu.VMEM((NB, BR), jnp.float32)],
    )(cls_pred, cls_pred, tgt3, tgt3)
    return out[0, 0]


def kernel(cls_pred, cls_target):
    tgt3 = cls_target.astype(jnp.int32).reshape(NB, 1, BR)
    return _ohem(cls_pred, tgt3)
